# baseline (speedup 1.0000x reference)
import functools

import jax
import jax.numpy as jnp
from jax import lax
from jax.experimental import pallas as pl
from jax.experimental.pallas import tpu as pltpu
from jax.experimental.pallas import tpu_sc as plsc

NC = 2
NS = 16
LANES = 16

CHUNK = 48
BLK_CH = 13
NBLK = 16


def _prep_body(params_ref, h_ref, w_ref, b_ref, whn_ref, wh_ref, whc_ref,
               invn_ref):
    wh = lax.dot_general(h_ref[...], w_ref[...], (((1,), (1,)), ((), ())),
                         preferred_element_type=jnp.float32) + b_ref[...]
    nrm = jnp.maximum(jnp.sqrt(jnp.sum(wh * wh, axis=1, keepdims=True)), 1e-8)
    whn_ref[...] = wh / nrm
    wh_ref[...] = wh
    whc_ref[...] = params_ref[0] * wh
    invn_ref[...] = 1.0 / nrm


def _prep_call(params, h, W, b2d):
    n, d = h.shape
    blk = 2000
    f32 = jnp.float32
    return pl.pallas_call(
        _prep_body,
        grid=(n // blk,),
        in_specs=[
            pl.BlockSpec(memory_space=pltpu.SMEM),
            pl.BlockSpec((blk, d), lambda i: (i, 0)),
            pl.BlockSpec((d, d), lambda i: (0, 0)),
            pl.BlockSpec((1, d), lambda i: (0, 0)),
        ],
        out_specs=[
            pl.BlockSpec((blk, d), lambda i: (i, 0)),
            pl.BlockSpec((blk, d), lambda i: (i, 0)),
            pl.BlockSpec((blk, d), lambda i: (i, 0)),
            pl.BlockSpec((blk, 1), lambda i: (i, 0)),
        ],
        out_shape=[
            jax.ShapeDtypeStruct((n, d), f32),
            jax.ShapeDtypeStruct((n, d), f32),
            jax.ShapeDtypeStruct((n, d), f32),
            jax.ShapeDtypeStruct((n, 1), f32),
        ],
    )(params, h, W, b2d)


def _ew_body(params_ref, src_ref, dst_ref, adj_ref, deg_ref, wpos_ref,
             wneg_ref):
    x = params_ref[0] * deg_ref[...] + params_ref[1]
    sp = jax.nn.softplus(x)
    base = jnp.where(src_ref[...] != dst_ref[...], adj_ref[...] * sp, 0.0)
    wpos_ref[...] = params_ref[2] * base
    wneg_ref[...] = params_ref[3] * base


def _ew_call(params, src2, dst2, adj2, deg2):
    f32 = jnp.float32
    return pl.pallas_call(
        _ew_body,
        in_specs=[
            pl.BlockSpec(memory_space=pltpu.SMEM),
            pl.BlockSpec(src2.shape, lambda: (0, 0)),
            pl.BlockSpec(dst2.shape, lambda: (0, 0)),
            pl.BlockSpec(adj2.shape, lambda: (0, 0)),
            pl.BlockSpec(adj2.shape, lambda: (0, 0)),
        ],
        out_specs=[
            pl.BlockSpec(adj2.shape, lambda: (0, 0)),
            pl.BlockSpec(adj2.shape, lambda: (0, 0)),
        ],
        out_shape=[
            jax.ShapeDtypeStruct(adj2.shape, f32),
            jax.ShapeDtypeStruct(adj2.shape, f32),
        ],
    )(params, src2, dst2, adj2, deg2)


def _hsum_bcast(x):
    idx = lax.iota(jnp.int32, LANES)
    for sh in (8, 4, 2, 1):
        x = x + x[idx ^ sh]
    return x


def _make_sc_edge(n, d):
    row_stride = 8 * (n // (8 * NS))
    row_width = n - (NS - 1) * row_stride
    f32 = jnp.float32
    i32 = jnp.int32
    nsub = d // LANES
    mesh = plsc.VectorSubcoreMesh(core_axis_name="c", subcore_axis_name="s")

    @functools.partial(
        pl.kernel,
        out_type=jax.ShapeDtypeStruct((NC, n, d), f32),
        mesh=mesh,
        compiler_params=pltpu.CompilerParams(
            needs_layout_passes=False,
            internal_scratch_in_bytes=64 * 1024,
        ),
        scratch_types=[
            pltpu.VMEM((BLK_CH, CHUNK), i32),
            pltpu.VMEM((BLK_CH, CHUNK), i32),
            pltpu.VMEM((BLK_CH, CHUNK), f32),
            pltpu.VMEM((BLK_CH, CHUNK), f32),
            pltpu.VMEM((n // 2,), i32),
            pltpu.VMEM((CHUNK, d), f32),
            pltpu.VMEM((CHUNK, d), f32),
            pltpu.VMEM((CHUNK, d), f32),
            pltpu.VMEM((CHUNK, d), f32),
            pltpu.VMEM_SHARED((n, d), f32),
            pltpu.SemaphoreType.DMA,
            pltpu.SemaphoreType.DMA,
            pltpu.SemaphoreType.DMA,
            pltpu.SemaphoreType.DMA,
            pltpu.SemaphoreType.DMA,
            pltpu.SemaphoreType.DMA,
        ],
    )
    def sc_edge(whn_hbm, wh_hbm, whc_hbm, src_hbm, dst_hbm, wpos_hbm,
                wneg_hbm, tsrc_hbm, tdst_hbm, twpos_hbm, twneg_hbm,
                invp_hbm, out_hbm,
                srcv, dstv, wposv, wnegv, invpv, sr0, sr1, dr0, dr1, acc,
                ss0, ss1, sd0, sd1, sc0, sc1):
        srows = (sr0, sr1)
        drows = (dr0, dr1)
        sems_s = (ss0, ss1)
        sems_d = (sd0, sd1)
        sems_c = (sc0, sc1)
        cid = lax.axis_index("c")
        sid = lax.axis_index("s")
        wid = cid * NS + sid
        row0 = sid * row_stride
        pltpu.sync_copy(whc_hbm.at[pl.ds(row0, row_width)],
                        acc.at[pl.ds(row0, row_width)])
        pltpu.sync_copy(invp_hbm, invpv)
        plsc.subcore_barrier()

        def issue_gathers(l, slot):
            pltpu.async_copy(whn_hbm.at[srcv.at[l]], srows[slot],
                             sems_s[slot])
            pltpu.async_copy(wh_hbm.at[dstv.at[l]], drows[slot],
                             sems_d[slot])

        def wait_gathers(slot):
            pltpu.make_async_copy(whn_hbm.at[srcv.at[0]], srows[slot],
                                  sems_s[slot]).wait()
            pltpu.make_async_copy(wh_hbm.at[dstv.at[0]], drows[slot],
                                  sems_d[slot]).wait()

        def drain_scatter(slot):
            pltpu.make_async_copy(drows[slot], acc.at[srcv.at[0]],
                                  sems_c[slot]).wait()

        def compute_chunk(l, slot):
            sref = srows[slot]
            dref = drows[slot]

            def group_body(g, carry2):
                g16 = g * LANES
                dst16 = dstv[l, pl.ds(g16, LANES)]
                packed = plsc.load_gather(
                    invpv, [lax.shift_right_logical(dst16, 1)])
                ea, eb = plsc.unpack(plsc.bitcast(packed, jnp.bfloat16),
                                     format=plsc.PackFormat.INTERLEAVED)
                invd = jnp.where((dst16 & 1) == 0, ea, eb)
                wp16 = wposv[l, pl.ds(g16, LANES)] * invd
                wn16 = wnegv[l, pl.ds(g16, LANES)] * invd
                for ee in range(LANES):
                    r = g16 + ee
                    vd = [dref[r, pl.ds(LANES * k, LANES)]
                          for k in range(nsub)]
                    p1 = None
                    for k in range(nsub):
                        vs = sref[r, pl.ds(LANES * k, LANES)]
                        t1 = vs * vd[k]
                        p1 = t1 if p1 is None else p1 + t1
                    dot1 = _hsum_bcast(p1)
                    lane = jnp.full((LANES,), ee, i32)
                    w = dot1 * jnp.where(dot1 > 0, wp16[lane], wn16[lane])
                    for k in range(nsub):
                        dref[r, pl.ds(LANES * k, LANES)] = vd[k] * w
                return carry2

            lax.fori_loop(0, CHUNK // LANES, group_body, 0)

        def block_body(m, carry):
            @pl.when(m != 0)
            def _():
                drain_scatter((BLK_CH - 1) % 2)

            pltpu.sync_copy(src_hbm.at[wid, m], srcv)
            pltpu.sync_copy(dst_hbm.at[wid, m], dstv)
            pltpu.sync_copy(wpos_hbm.at[wid, m], wposv)
            pltpu.sync_copy(wneg_hbm.at[wid, m], wnegv)
            issue_gathers(0, 0)

            def pair_body(ii, carry1):
                for b in range(2):
                    l = ii * 2 + b
                    wait_gathers(b)
                    if b == 1:
                        drain_scatter(0)
                    else:
                        @pl.when(ii != 0)
                        def _():
                            drain_scatter(1)
                    issue_gathers(l + 1, 1 - b)
                    compute_chunk(l, b)
                    pltpu.async_copy(drows[b], acc.at[srcv.at[l]],
                                     sems_c[b], add=True)
                return carry1

            lax.fori_loop(0, (BLK_CH - 1) // 2, pair_body, 0)
            lf = BLK_CH - 1
            wait_gathers(lf % 2)
            drain_scatter(1 - lf % 2)
            compute_chunk(lf, lf % 2)
            pltpu.async_copy(drows[lf % 2], acc.at[srcv.at[lf]],
                             sems_c[lf % 2], add=True)
            return carry

        lax.fori_loop(0, NBLK, block_body, 0)
        drain_scatter((BLK_CH - 1) % 2)

        pltpu.sync_copy(tsrc_hbm.at[wid], srcv.at[pl.ds(0, 1)])
        pltpu.sync_copy(tdst_hbm.at[wid], dstv.at[pl.ds(0, 1)])
        pltpu.sync_copy(twpos_hbm.at[wid], wposv.at[pl.ds(0, 1)])
        pltpu.sync_copy(twneg_hbm.at[wid], wnegv.at[pl.ds(0, 1)])
        issue_gathers(0, 0)
        wait_gathers(0)
        compute_chunk(0, 0)
        pltpu.sync_copy(drows[0], acc.at[srcv.at[0]], add=True)

        plsc.subcore_barrier()
        pltpu.sync_copy(acc.at[pl.ds(row0, row_width)],
                        out_hbm.at[cid, pl.ds(row0, row_width)])

    return sc_edge


def _comb_body(a_ref, b_ref, o_ref):
    o_ref[...] = a_ref[...] + b_ref[...]


def _comb_call(a, b):
    n, d = a.shape
    blk = 2000
    return pl.pallas_call(
        _comb_body,
        grid=(n // blk,),
        in_specs=[
            pl.BlockSpec((blk, d), lambda i: (i, 0)),
            pl.BlockSpec((blk, d), lambda i: (i, 0)),
        ],
        out_specs=pl.BlockSpec((blk, d), lambda i: (i, 0)),
        out_shape=jax.ShapeDtypeStruct((n, d), jnp.float32),
    )(a, b)


def kernel(h, edge_index, adj_values, deg_values, W, b, deg_coeff, coeff,
           scale):
    n, d = h.shape
    e = edge_index.shape[1]
    c = jax.nn.softmax(coeff)
    s = jax.nn.softplus(scale)[0]
    cp = s * c[0]
    cn = s * c[1]
    cc2 = 0.5 * s * c[2]

    params_a = jnp.reshape(cc2, (1,))
    Whn, Wh, Whc, invn = _prep_call(params_a, h, W, b.reshape(1, d))
    invp = lax.bitcast_convert_type(
        invn.reshape(n).astype(jnp.bfloat16).reshape(n // 2, 2),
        jnp.int32).reshape(n // 2)

    src = edge_index[0]
    dst = edge_index[1]
    ew_cols = 128
    params_b = jnp.stack([deg_coeff[0], deg_coeff[1], cp, cn])
    wpos, wneg = _ew_call(params_b,
                          src.reshape(e // ew_cols, ew_cols),
                          dst.reshape(e // ew_cols, ew_cols),
                          adj_values.reshape(e // ew_cols, ew_cols),
                          deg_values.reshape(e // ew_cols, ew_cols))

    nw = NC * NS
    epw = e // nw
    main = NBLK * BLK_CH * CHUNK
    t0 = epw - CHUNK
    mshape = (nw, NBLK, BLK_CH, CHUNK)

    srcw = src.reshape(nw, epw)
    dstw = dst.reshape(nw, epw)
    wposw = wpos.reshape(nw, epw)
    wnegw = wneg.reshape(nw, epw)
    tmask = (jnp.arange(CHUNK) >= (main - t0)).astype(jnp.float32)

    sc_edge = _make_sc_edge(n, d)
    partial = sc_edge(
        Whn, Wh, Whc,
        srcw[:, :main].reshape(mshape),
        dstw[:, :main].reshape(mshape),
        wposw[:, :main].reshape(mshape),
        wnegw[:, :main].reshape(mshape),
        srcw[:, t0:].reshape(nw, 1, CHUNK),
        dstw[:, t0:].reshape(nw, 1, CHUNK),
        (wposw[:, t0:] * tmask).reshape(nw, 1, CHUNK),
        (wnegw[:, t0:] * tmask).reshape(nw, 1, CHUNK),
        invp)

    return _comb_call(partial[0], partial[1])


# speedup vs baseline: 1.0510x; 1.0510x over previous
import functools

import jax
import jax.numpy as jnp
from jax import lax
from jax.experimental import pallas as pl
from jax.experimental.pallas import tpu as pltpu
from jax.experimental.pallas import tpu_sc as plsc

NC = 2
NS = 16
LANES = 16

CHUNK = 64
BLK_CH = 13
NBLK = 12


def _prep_body(params_ref, h_ref, w_ref, b_ref, whn_ref, wh_ref, whc_ref,
               invn_ref):
    wh = lax.dot_general(h_ref[...], w_ref[...], (((1,), (1,)), ((), ())),
                         preferred_element_type=jnp.float32) + b_ref[...]
    nrm = jnp.maximum(jnp.sqrt(jnp.sum(wh * wh, axis=1, keepdims=True)), 1e-8)
    whn_ref[...] = wh / nrm
    wh_ref[...] = wh
    whc_ref[...] = params_ref[0] * wh
    invn_ref[...] = 1.0 / nrm


def _prep_call(params, h, W, b2d):
    n, d = h.shape
    blk = 2000
    f32 = jnp.float32
    return pl.pallas_call(
        _prep_body,
        grid=(n // blk,),
        in_specs=[
            pl.BlockSpec(memory_space=pltpu.SMEM),
            pl.BlockSpec((blk, d), lambda i: (i, 0)),
            pl.BlockSpec((d, d), lambda i: (0, 0)),
            pl.BlockSpec((1, d), lambda i: (0, 0)),
        ],
        out_specs=[
            pl.BlockSpec((blk, d), lambda i: (i, 0)),
            pl.BlockSpec((blk, d), lambda i: (i, 0)),
            pl.BlockSpec((blk, d), lambda i: (i, 0)),
            pl.BlockSpec((blk, 1), lambda i: (i, 0)),
        ],
        out_shape=[
            jax.ShapeDtypeStruct((n, d), f32),
            jax.ShapeDtypeStruct((n, d), f32),
            jax.ShapeDtypeStruct((n, d), f32),
            jax.ShapeDtypeStruct((n, 1), f32),
        ],
    )(params, h, W, b2d)


def _ew_body(params_ref, src_ref, dst_ref, adj_ref, deg_ref, wpos_ref,
             wneg_ref):
    x = params_ref[0] * deg_ref[...] + params_ref[1]
    sp = jax.nn.softplus(x)
    base = jnp.where(src_ref[...] != dst_ref[...], adj_ref[...] * sp, 0.0)
    wpos_ref[...] = params_ref[2] * base
    wneg_ref[...] = params_ref[3] * base


def _ew_call(params, src2, dst2, adj2, deg2):
    f32 = jnp.float32
    return pl.pallas_call(
        _ew_body,
        in_specs=[
            pl.BlockSpec(memory_space=pltpu.SMEM),
            pl.BlockSpec(src2.shape, lambda: (0, 0)),
            pl.BlockSpec(dst2.shape, lambda: (0, 0)),
            pl.BlockSpec(adj2.shape, lambda: (0, 0)),
            pl.BlockSpec(adj2.shape, lambda: (0, 0)),
        ],
        out_specs=[
            pl.BlockSpec(adj2.shape, lambda: (0, 0)),
            pl.BlockSpec(adj2.shape, lambda: (0, 0)),
        ],
        out_shape=[
            jax.ShapeDtypeStruct(adj2.shape, f32),
            jax.ShapeDtypeStruct(adj2.shape, f32),
        ],
    )(params, src2, dst2, adj2, deg2)


def _hsum_bcast(x):
    idx = lax.iota(jnp.int32, LANES)
    for sh in (8, 4, 2, 1):
        x = x + x[idx ^ sh]
    return x


def _rsqrt_vec(x):
    xi = plsc.bitcast(x, jnp.int32)
    yi = jnp.int32(0x5F3759DF) - lax.shift_right_logical(xi, 1)
    y = plsc.bitcast(yi, jnp.float32)
    half = -0.5 * x
    y = y * (1.5 + half * y * y)
    y = y * (1.5 + half * y * y)
    return y


def _make_sc_edge(n, d):
    row_stride = 8 * (n // (8 * NS))
    row_width = n - (NS - 1) * row_stride
    f32 = jnp.float32
    i32 = jnp.int32
    nsub = d // LANES
    mesh = plsc.VectorSubcoreMesh(core_axis_name="c", subcore_axis_name="s")

    @functools.partial(
        pl.kernel,
        out_type=jax.ShapeDtypeStruct((NC, n, d), f32),
        mesh=mesh,
        compiler_params=pltpu.CompilerParams(
            needs_layout_passes=False,
            internal_scratch_in_bytes=64 * 1024,
        ),
        scratch_types=[
            pltpu.VMEM((BLK_CH, CHUNK), i32),
            pltpu.VMEM((BLK_CH, CHUNK), i32),
            pltpu.VMEM((BLK_CH, CHUNK), f32),
            pltpu.VMEM((BLK_CH, CHUNK), f32),
            pltpu.VMEM((CHUNK, d), f32),
            pltpu.VMEM((CHUNK, d), f32),
            pltpu.VMEM((CHUNK, d), f32),
            pltpu.VMEM((CHUNK, d), f32),
            pltpu.VMEM_SHARED((n, d), f32),
            pltpu.SemaphoreType.DMA,
            pltpu.SemaphoreType.DMA,
            pltpu.SemaphoreType.DMA,
            pltpu.SemaphoreType.DMA,
            pltpu.SemaphoreType.DMA,
            pltpu.SemaphoreType.DMA,
        ],
    )
    def sc_edge(whn_hbm, wh_hbm, whc_hbm, src_hbm, dst_hbm, wpos_hbm,
                wneg_hbm, tsrc_hbm, tdst_hbm, twpos_hbm, twneg_hbm, out_hbm,
                srcv, dstv, wposv, wnegv, sr0, sr1, dr0, dr1, acc,
                ss0, ss1, sd0, sd1, sc0, sc1):
        srows = (sr0, sr1)
        drows = (dr0, dr1)
        sems_s = (ss0, ss1)
        sems_d = (sd0, sd1)
        sems_c = (sc0, sc1)
        cid = lax.axis_index("c")
        sid = lax.axis_index("s")
        wid = cid * NS + sid
        row0 = sid * row_stride
        pltpu.sync_copy(whc_hbm.at[pl.ds(row0, row_width)],
                        acc.at[pl.ds(row0, row_width)])
        plsc.subcore_barrier()

        def issue_gathers(l, slot):
            pltpu.async_copy(whn_hbm.at[srcv.at[l]], srows[slot],
                             sems_s[slot])
            pltpu.async_copy(wh_hbm.at[dstv.at[l]], drows[slot],
                             sems_d[slot])

        def wait_gathers(slot):
            pltpu.make_async_copy(whn_hbm.at[srcv.at[0]], srows[slot],
                                  sems_s[slot]).wait()
            pltpu.make_async_copy(wh_hbm.at[dstv.at[0]], drows[slot],
                                  sems_d[slot]).wait()

        def drain_scatter(slot):
            pltpu.make_async_copy(drows[slot], acc.at[srcv.at[0]],
                                  sems_c[slot]).wait()

        def compute_chunk(l, slot):
            sref = srows[slot]
            dref = drows[slot]

            def group_body(g, carry2):
                g16 = g * LANES
                wp16 = wposv[l, pl.ds(g16, LANES)]
                wn16 = wnegv[l, pl.ds(g16, LANES)]
                for ee in range(LANES):
                    r = g16 + ee
                    vd = [dref[r, pl.ds(LANES * k, LANES)]
                          for k in range(nsub)]
                    p1 = None
                    p2 = None
                    for k in range(nsub):
                        vs = sref[r, pl.ds(LANES * k, LANES)]
                        t1 = vs * vd[k]
                        t2 = vd[k] * vd[k]
                        p1 = t1 if p1 is None else p1 + t1
                        p2 = t2 if p2 is None else p2 + t2
                    dot1 = _hsum_bcast(p1)
                    dot2 = jnp.maximum(_hsum_bcast(p2), 1e-16)
                    sim = dot1 * _rsqrt_vec(dot2)
                    lane = jnp.full((LANES,), ee, i32)
                    w = sim * jnp.where(sim > 0, wp16[lane], wn16[lane])
                    for k in range(nsub):
                        dref[r, pl.ds(LANES * k, LANES)] = vd[k] * w
                return carry2

            lax.fori_loop(0, CHUNK // LANES, group_body, 0)

        def block_body(m, carry):
            @pl.when(m != 0)
            def _():
                drain_scatter((BLK_CH - 1) % 2)

            pltpu.sync_copy(src_hbm.at[wid, m], srcv)
            pltpu.sync_copy(dst_hbm.at[wid, m], dstv)
            pltpu.sync_copy(wpos_hbm.at[wid, m], wposv)
            pltpu.sync_copy(wneg_hbm.at[wid, m], wnegv)
            issue_gathers(0, 0)

            def pair_body(ii, carry1):
                for b in range(2):
                    l = ii * 2 + b
                    wait_gathers(b)
                    if b == 1:
                        drain_scatter(0)
                    else:
                        @pl.when(ii != 0)
                        def _():
                            drain_scatter(1)
                    issue_gathers(l + 1, 1 - b)
                    compute_chunk(l, b)
                    pltpu.async_copy(drows[b], acc.at[srcv.at[l]],
                                     sems_c[b], add=True)
                return carry1

            lax.fori_loop(0, (BLK_CH - 1) // 2, pair_body, 0)
            lf = BLK_CH - 1
            wait_gathers(lf % 2)
            drain_scatter(1 - lf % 2)
            compute_chunk(lf, lf % 2)
            pltpu.async_copy(drows[lf % 2], acc.at[srcv.at[lf]],
                             sems_c[lf % 2], add=True)
            return carry

        lax.fori_loop(0, NBLK, block_body, 0)
        drain_scatter((BLK_CH - 1) % 2)

        pltpu.sync_copy(tsrc_hbm.at[wid], srcv.at[pl.ds(0, 1)])
        pltpu.sync_copy(tdst_hbm.at[wid], dstv.at[pl.ds(0, 1)])
        pltpu.sync_copy(twpos_hbm.at[wid], wposv.at[pl.ds(0, 1)])
        pltpu.sync_copy(twneg_hbm.at[wid], wnegv.at[pl.ds(0, 1)])
        issue_gathers(0, 0)
        wait_gathers(0)
        compute_chunk(0, 0)
        pltpu.sync_copy(drows[0], acc.at[srcv.at[0]], add=True)

        plsc.subcore_barrier()
        pltpu.sync_copy(acc.at[pl.ds(row0, row_width)],
                        out_hbm.at[cid, pl.ds(row0, row_width)])

    return sc_edge


def _comb_body(a_ref, b_ref, o_ref):
    o_ref[...] = a_ref[...] + b_ref[...]


def _comb_call(a, b):
    n, d = a.shape
    blk = 2000
    return pl.pallas_call(
        _comb_body,
        grid=(n // blk,),
        in_specs=[
            pl.BlockSpec((blk, d), lambda i: (i, 0)),
            pl.BlockSpec((blk, d), lambda i: (i, 0)),
        ],
        out_specs=pl.BlockSpec((blk, d), lambda i: (i, 0)),
        out_shape=jax.ShapeDtypeStruct((n, d), jnp.float32),
    )(a, b)


def kernel(h, edge_index, adj_values, deg_values, W, b, deg_coeff, coeff,
           scale):
    n, d = h.shape
    e = edge_index.shape[1]
    c = jax.nn.softmax(coeff)
    s = jax.nn.softplus(scale)[0]
    cp = s * c[0]
    cn = s * c[1]
    cc2 = 0.5 * s * c[2]

    params_a = jnp.reshape(cc2, (1,))
    Whn, Wh, Whc, _invn = _prep_call(params_a, h, W, b.reshape(1, d))

    src = edge_index[0]
    dst = edge_index[1]
    ew_cols = 128
    params_b = jnp.stack([deg_coeff[0], deg_coeff[1], cp, cn])
    wpos, wneg = _ew_call(params_b,
                          src.reshape(e // ew_cols, ew_cols),
                          dst.reshape(e // ew_cols, ew_cols),
                          adj_values.reshape(e // ew_cols, ew_cols),
                          deg_values.reshape(e // ew_cols, ew_cols))

    nw = NC * NS
    epw = e // nw
    main = NBLK * BLK_CH * CHUNK
    t0 = epw - CHUNK
    mshape = (nw, NBLK, BLK_CH, CHUNK)

    srcw = src.reshape(nw, epw)
    dstw = dst.reshape(nw, epw)
    wposw = wpos.reshape(nw, epw)
    wnegw = wneg.reshape(nw, epw)
    tmask = (jnp.arange(CHUNK) >= (main - t0)).astype(jnp.float32)

    sc_edge = _make_sc_edge(n, d)
    partial = sc_edge(
        Whn, Wh, Whc,
        srcw[:, :main].reshape(mshape),
        dstw[:, :main].reshape(mshape),
        wposw[:, :main].reshape(mshape),
        wnegw[:, :main].reshape(mshape),
        srcw[:, t0:].reshape(nw, 1, CHUNK),
        dstw[:, t0:].reshape(nw, 1, CHUNK),
        (wposw[:, t0:] * tmask).reshape(nw, 1, CHUNK),
        (wnegw[:, t0:] * tmask).reshape(nw, 1, CHUNK))

    return _comb_call(partial[0], partial[1])


# speedup vs baseline: 1.0529x; 1.0018x over previous
import functools

import jax
import jax.numpy as jnp
from jax import lax
from jax.experimental import pallas as pl
from jax.experimental.pallas import tpu as pltpu
from jax.experimental.pallas import tpu_sc as plsc

NC = 2
NS = 16
LANES = 16

CHUNK = 64
BLK_CH = 13
NBLK = 12


def _prep_body(params_ref, h_ref, w_ref, b_ref, whn_ref, wh_ref, whc_ref):
    wh = lax.dot_general(h_ref[...], w_ref[...], (((1,), (1,)), ((), ())),
                         preferred_element_type=jnp.float32) + b_ref[...]
    nrm = jnp.maximum(jnp.sqrt(jnp.sum(wh * wh, axis=1, keepdims=True)), 1e-8)
    whn_ref[...] = wh / nrm
    wh_ref[...] = wh
    whc_ref[...] = params_ref[0] * wh


def _prep_call(params, h, W, b2d):
    n, d = h.shape
    blk = 2000
    f32 = jnp.float32
    return pl.pallas_call(
        _prep_body,
        grid=(n // blk,),
        in_specs=[
            pl.BlockSpec(memory_space=pltpu.SMEM),
            pl.BlockSpec((blk, d), lambda i: (i, 0)),
            pl.BlockSpec((d, d), lambda i: (0, 0)),
            pl.BlockSpec((1, d), lambda i: (0, 0)),
        ],
        out_specs=[
            pl.BlockSpec((blk, d), lambda i: (i, 0)),
            pl.BlockSpec((blk, d), lambda i: (i, 0)),
            pl.BlockSpec((blk, d), lambda i: (i, 0)),
        ],
        out_shape=[
            jax.ShapeDtypeStruct((n, d), f32),
            jax.ShapeDtypeStruct((n, d), f32),
            jax.ShapeDtypeStruct((n, d), f32),
        ],
    )(params, h, W, b2d)


def _ew_body(params_ref, src_ref, dst_ref, adj_ref, deg_ref, wpos_ref,
             wneg_ref):
    x = params_ref[0] * deg_ref[...] + params_ref[1]
    sp = jax.nn.softplus(x)
    base = jnp.where(src_ref[...] != dst_ref[...], adj_ref[...] * sp, 0.0)
    wpos_ref[...] = params_ref[2] * base
    wneg_ref[...] = params_ref[3] * base


def _ew_call(params, src2, dst2, adj2, deg2):
    f32 = jnp.float32
    return pl.pallas_call(
        _ew_body,
        in_specs=[
            pl.BlockSpec(memory_space=pltpu.SMEM),
            pl.BlockSpec(src2.shape, lambda: (0, 0)),
            pl.BlockSpec(dst2.shape, lambda: (0, 0)),
            pl.BlockSpec(adj2.shape, lambda: (0, 0)),
            pl.BlockSpec(adj2.shape, lambda: (0, 0)),
        ],
        out_specs=[
            pl.BlockSpec(adj2.shape, lambda: (0, 0)),
            pl.BlockSpec(adj2.shape, lambda: (0, 0)),
        ],
        out_shape=[
            jax.ShapeDtypeStruct(adj2.shape, f32),
            jax.ShapeDtypeStruct(adj2.shape, f32),
        ],
    )(params, src2, dst2, adj2, deg2)


def _hsum_bcast(x):
    idx = lax.iota(jnp.int32, LANES)
    for sh in (8, 4, 2, 1):
        x = x + x[idx ^ sh]
    return x


def _rsqrt_vec(x):
    xi = plsc.bitcast(x, jnp.int32)
    yi = jnp.int32(0x5F3759DF) - lax.shift_right_logical(xi, 1)
    y = plsc.bitcast(yi, jnp.float32)
    half = -0.5 * x
    y = y * (1.5 + half * y * y)
    y = y * (1.5 + half * y * y)
    return y


def _make_sc_edge(n, d):
    row_stride = 8 * (n // (8 * NS))
    row_width = n - (NS - 1) * row_stride
    f32 = jnp.float32
    i32 = jnp.int32
    nsub = d // LANES
    mesh = plsc.VectorSubcoreMesh(core_axis_name="c", subcore_axis_name="s")

    @functools.partial(
        pl.kernel,
        out_type=jax.ShapeDtypeStruct((NC, n, d), f32),
        mesh=mesh,
        compiler_params=pltpu.CompilerParams(
            needs_layout_passes=False,
            internal_scratch_in_bytes=64 * 1024,
        ),
        scratch_types=[
            pltpu.VMEM((BLK_CH, CHUNK), i32),
            pltpu.VMEM((BLK_CH, CHUNK), i32),
            pltpu.VMEM((BLK_CH, CHUNK), f32),
            pltpu.VMEM((BLK_CH, CHUNK), f32),
            pltpu.VMEM((CHUNK, d), f32),
            pltpu.VMEM((CHUNK, d), f32),
            pltpu.VMEM((CHUNK, d), f32),
            pltpu.VMEM((CHUNK, d), f32),
            pltpu.VMEM_SHARED((n, d), f32),
            pltpu.SemaphoreType.DMA,
            pltpu.SemaphoreType.DMA,
            pltpu.SemaphoreType.DMA,
            pltpu.SemaphoreType.DMA,
            pltpu.SemaphoreType.DMA,
            pltpu.SemaphoreType.DMA,
        ],
    )
    def sc_edge(whn_hbm, wh_hbm, whc_hbm, src_hbm, dst_hbm, wpos_hbm,
                wneg_hbm, tsrc_hbm, tdst_hbm, twpos_hbm, twneg_hbm, out_hbm,
                srcv, dstv, wposv, wnegv, sr0, sr1, dr0, dr1, acc,
                ss0, ss1, sd0, sd1, sc0, sc1):
        srows = (sr0, sr1)
        drows = (dr0, dr1)
        sems_s = (ss0, ss1)
        sems_d = (sd0, sd1)
        sems_c = (sc0, sc1)
        cid = lax.axis_index("c")
        sid = lax.axis_index("s")
        wid = cid * NS + sid
        row0 = sid * row_stride
        pltpu.sync_copy(whc_hbm.at[pl.ds(row0, row_width)],
                        acc.at[pl.ds(row0, row_width)])
        plsc.subcore_barrier()

        def issue_gathers(l, slot):
            pltpu.async_copy(whn_hbm.at[srcv.at[l]], srows[slot],
                             sems_s[slot])
            pltpu.async_copy(wh_hbm.at[dstv.at[l]], drows[slot],
                             sems_d[slot])

        def wait_gathers(slot):
            pltpu.make_async_copy(whn_hbm.at[srcv.at[0]], srows[slot],
                                  sems_s[slot]).wait()
            pltpu.make_async_copy(wh_hbm.at[dstv.at[0]], drows[slot],
                                  sems_d[slot]).wait()

        def drain_scatter(slot):
            pltpu.make_async_copy(drows[slot], acc.at[srcv.at[0]],
                                  sems_c[slot]).wait()

        def compute_chunk(l, slot):
            sref = srows[slot]
            dref = drows[slot]

            def group_body(g, carry2):
                g16 = g * LANES
                wp16 = wposv[l, pl.ds(g16, LANES)]
                wn16 = wnegv[l, pl.ds(g16, LANES)]
                for ee in range(LANES):
                    r = g16 + ee
                    vd = [dref[r, pl.ds(LANES * k, LANES)]
                          for k in range(nsub)]
                    p1 = None
                    p2 = None
                    for k in range(nsub):
                        vs = sref[r, pl.ds(LANES * k, LANES)]
                        t1 = vs * vd[k]
                        t2 = vd[k] * vd[k]
                        p1 = t1 if p1 is None else p1 + t1
                        p2 = t2 if p2 is None else p2 + t2
                    dot1 = _hsum_bcast(p1)
                    dot2 = jnp.maximum(_hsum_bcast(p2), 1e-16)
                    sim = dot1 * _rsqrt_vec(dot2)
                    lane = jnp.full((LANES,), ee, i32)
                    w = sim * jnp.where(sim > 0, wp16[lane], wn16[lane])
                    for k in range(nsub):
                        dref[r, pl.ds(LANES * k, LANES)] = vd[k] * w
                return carry2

            lax.fori_loop(0, CHUNK // LANES, group_body, 0)

        def block_body(m, carry):
            @pl.when(m != 0)
            def _():
                drain_scatter((BLK_CH - 1) % 2)

            pltpu.sync_copy(src_hbm.at[wid, m], srcv)
            pltpu.sync_copy(dst_hbm.at[wid, m], dstv)
            pltpu.sync_copy(wpos_hbm.at[wid, m], wposv)
            pltpu.sync_copy(wneg_hbm.at[wid, m], wnegv)
            issue_gathers(0, 0)

            def pair_body(ii, carry1):
                for b in range(2):
                    l = ii * 2 + b
                    wait_gathers(b)
                    if b == 1:
                        drain_scatter(0)
                    else:
                        @pl.when(ii != 0)
                        def _():
                            drain_scatter(1)
                    issue_gathers(l + 1, 1 - b)
                    compute_chunk(l, b)
                    pltpu.async_copy(drows[b], acc.at[srcv.at[l]],
                                     sems_c[b], add=True)
                return carry1

            lax.fori_loop(0, (BLK_CH - 1) // 2, pair_body, 0)
            lf = BLK_CH - 1
            wait_gathers(lf % 2)
            drain_scatter(1 - lf % 2)
            compute_chunk(lf, lf % 2)
            pltpu.async_copy(drows[lf % 2], acc.at[srcv.at[lf]],
                             sems_c[lf % 2], add=True)
            return carry

        lax.fori_loop(0, NBLK, block_body, 0)
        drain_scatter((BLK_CH - 1) % 2)

        pltpu.sync_copy(tsrc_hbm.at[wid], srcv.at[pl.ds(0, 1)])
        pltpu.sync_copy(tdst_hbm.at[wid], dstv.at[pl.ds(0, 1)])
        pltpu.sync_copy(twpos_hbm.at[wid], wposv.at[pl.ds(0, 1)])
        pltpu.sync_copy(twneg_hbm.at[wid], wnegv.at[pl.ds(0, 1)])
        issue_gathers(0, 0)
        wait_gathers(0)
        compute_chunk(0, 0)
        pltpu.sync_copy(drows[0], acc.at[srcv.at[0]], add=True)

        plsc.subcore_barrier()
        pltpu.sync_copy(acc.at[pl.ds(row0, row_width)],
                        out_hbm.at[cid, pl.ds(row0, row_width)])

    return sc_edge


def _comb_body(a_ref, b_ref, o_ref):
    o_ref[...] = a_ref[...] + b_ref[...]


def _comb_call(a, b):
    n, d = a.shape
    blk = 2000
    return pl.pallas_call(
        _comb_body,
        grid=(n // blk,),
        in_specs=[
            pl.BlockSpec((blk, d), lambda i: (i, 0)),
            pl.BlockSpec((blk, d), lambda i: (i, 0)),
        ],
        out_specs=pl.BlockSpec((blk, d), lambda i: (i, 0)),
        out_shape=jax.ShapeDtypeStruct((n, d), jnp.float32),
    )(a, b)


def kernel(h, edge_index, adj_values, deg_values, W, b, deg_coeff, coeff,
           scale):
    n, d = h.shape
    e = edge_index.shape[1]
    c = jax.nn.softmax(coeff)
    s = jax.nn.softplus(scale)[0]
    cp = s * c[0]
    cn = s * c[1]
    cc2 = 0.5 * s * c[2]

    params_a = jnp.reshape(cc2, (1,))
    Whn, Wh, Whc = _prep_call(params_a, h, W, b.reshape(1, d))

    src = edge_index[0]
    dst = edge_index[1]
    ew_cols = 128
    params_b = jnp.stack([deg_coeff[0], deg_coeff[1], cp, cn])
    wpos, wneg = _ew_call(params_b,
                          src.reshape(e // ew_cols, ew_cols),
                          dst.reshape(e // ew_cols, ew_cols),
                          adj_values.reshape(e // ew_cols, ew_cols),
                          deg_values.reshape(e // ew_cols, ew_cols))

    nw = NC * NS
    epw = e // nw
    main = NBLK * BLK_CH * CHUNK
    t0 = epw - CHUNK
    mshape = (nw, NBLK, BLK_CH, CHUNK)

    srcw = src.reshape(nw, epw)
    dstw = dst.reshape(nw, epw)
    wposw = wpos.reshape(nw, epw)
    wnegw = wneg.reshape(nw, epw)
    tmask = (jnp.arange(CHUNK) >= (main - t0)).astype(jnp.float32)

    sc_edge = _make_sc_edge(n, d)
    partial = sc_edge(
        Whn, Wh, Whc,
        srcw[:, :main].reshape(mshape),
        dstw[:, :main].reshape(mshape),
        wposw[:, :main].reshape(mshape),
        wnegw[:, :main].reshape(mshape),
        srcw[:, t0:].reshape(nw, 1, CHUNK),
        dstw[:, t0:].reshape(nw, 1, CHUNK),
        (wposw[:, t0:] * tmask).reshape(nw, 1, CHUNK),
        (wnegw[:, t0:] * tmask).reshape(nw, 1, CHUNK))

    return _comb_call(partial[0], partial[1])
